# baseline (speedup 1.0000x reference)
import jax
import jax.numpy as jnp
from jax import lax
from jax.experimental import pallas as pl
from jax.experimental.pallas import tpu as pltpu

B, N, D = 128, 576, 768
CB = 8
NSTEP = B // CB
NBUF = 4


def _ring_kernel(x_hbm, t_hbm, o_hbm, tbl,
                 v0, v1, v2, v3, ts, si0, si1, si2, si3, so0, so1, so2, so3):
    slots = (v0, v1, v2, v3)
    sins = (si0, si1, si2, si3)
    souts = (so0, so1, so2, so3)

    pltpu.async_copy(t_hbm, tbl, ts).wait()

    def src(s):
        return x_hbm.at[pl.ds(s * CB, CB)]

    def dst(s):
        return o_hbm.at[pl.ds(s * CB, CB)]

    def add(slot):
        slot[...] = slot[...] + tbl[...][None, :, :]

    pltpu.async_copy(src(0), slots[0], sins[0])
    pltpu.async_copy(src(1), slots[1], sins[1])
    for s in (0, 1):
        pltpu.make_async_copy(src(s), slots[s], sins[s]).wait()
        pltpu.async_copy(src(s + 2), slots[s + 2], sins[s + 2])
        add(slots[s])
        pltpu.async_copy(slots[s], dst(s), souts[s])

    def group(g, c):
        for k in range(4):
            s = 4 * g + 2 + k
            i = (2 + k) % 4
            j = k % 4
            pltpu.make_async_copy(src(s), slots[i], sins[i]).wait()
            pltpu.make_async_copy(slots[j], dst(s - 2), souts[j]).wait()
            pltpu.async_copy(src(s + 2), slots[j], sins[j])
            add(slots[i])
            pltpu.async_copy(slots[i], dst(s), souts[i])
        return c

    lax.fori_loop(0, (NSTEP - 4) // 4, group, 0)

    for s in (NSTEP - 2, NSTEP - 1):
        i = s % 4
        pltpu.make_async_copy(src(s), slots[i], sins[i]).wait()
        pltpu.make_async_copy(slots[(s + 2) % 4], dst(s - 2), souts[(s + 2) % 4]).wait()
        add(slots[i])
        pltpu.async_copy(slots[i], dst(s), souts[i])
    for s in (NSTEP - 2, NSTEP - 1):
        i = s % 4
        pltpu.make_async_copy(slots[i], dst(s), souts[i]).wait()


def kernel(encoded_patches, position_embedding):
    return pl.pallas_call(
        _ring_kernel,
        in_specs=[
            pl.BlockSpec(memory_space=pltpu.HBM),
            pl.BlockSpec(memory_space=pltpu.HBM),
        ],
        out_specs=pl.BlockSpec(memory_space=pltpu.HBM),
        out_shape=jax.ShapeDtypeStruct((B, N, D), jnp.float32),
        scratch_shapes=(
            [pltpu.VMEM((N, D), jnp.float32)]
            + [pltpu.VMEM((CB, N, D), jnp.float32) for _ in range(NBUF)]
            + [pltpu.SemaphoreType.DMA for _ in range(2 * NBUF + 1)]
        ),
        compiler_params=pltpu.CompilerParams(
            vmem_limit_bytes=62 * 1024 * 1024,
        ),
    )(encoded_patches, position_embedding)
